# baseline (speedup 1.0000x reference)
import jax
import jax.numpy as jnp
from jax import lax
from jax.experimental import pallas as pl
from jax.experimental.pallas import tpu as pltpu
from jax.experimental.pallas import tpu_sc as plsc

B, F, V = 16384, 26, 1000000
F2 = 32
BF2 = B * F2
NC, NS = 2, 16
NW = NC * NS
E = BF2 // NW

XR = BF2 // 128
NBLK = 8
BR = XR // NBLK
SPR = 128 // F2


def _sc_gather_body(idx_hbm, w_hbm, g_hbm, idx_v, g_v, sem):
    wid = lax.axis_index("s") * NC + lax.axis_index("c")
    base = wid * E
    pltpu.sync_copy(idx_hbm.at[pl.ds(base, E)], idx_v)
    pltpu.async_copy(w_hbm.at[idx_v], g_v, sem).wait()
    pltpu.sync_copy(g_v, g_hbm.at[pl.ds(base, E)])


def _tc_reduce_body(b_ref, g_ref, v_ref, m_ref, o_ref):
    p = g_ref[...] * v_ref[...]
    acc = jnp.dot(p, m_ref[...], preferred_element_type=jnp.float32)
    o_ref[...] = jax.nn.sigmoid(acc + b_ref[0])


def kernel(feature_idx, feature_vals, W, b):
    idx32 = jnp.pad(feature_idx.astype(jnp.int32),
                    ((0, 0), (0, F2 - F))).reshape(BF2)
    vals32 = jnp.pad(feature_vals, ((0, 0), (0, F2 - F))).reshape(BF2)

    mesh = plsc.VectorSubcoreMesh(core_axis_name="c", subcore_axis_name="s")
    gathered = pl.kernel(
        _sc_gather_body,
        out_type=jax.ShapeDtypeStruct((BF2,), jnp.float32),
        mesh=mesh,
        scratch_types=[
            pltpu.VMEM((E,), jnp.int32),
            pltpu.VMEM((E,), jnp.float32),
            pltpu.SemaphoreType.DMA,
        ],
    )(idx32, W)

    c = jnp.arange(128, dtype=jnp.int32)[:, None]
    s = jnp.arange(SPR, dtype=jnp.int32)[None, :]
    m = (c // F2 == s).astype(jnp.float32)

    out2d = pl.pallas_call(
        _tc_reduce_body,
        grid=(NBLK,),
        in_specs=[
            pl.BlockSpec(memory_space=pltpu.SMEM),
            pl.BlockSpec((BR, 128), lambda i: (i, 0)),
            pl.BlockSpec((BR, 128), lambda i: (i, 0)),
            pl.BlockSpec((128, SPR), lambda i: (0, 0)),
        ],
        out_specs=pl.BlockSpec((BR, SPR), lambda i: (i, 0)),
        out_shape=jax.ShapeDtypeStruct((XR, SPR), jnp.float32),
    )(jnp.asarray(b, jnp.float32).reshape(1),
      gathered.reshape(XR, 128), vals32.reshape(XR, 128), m)
    return out2d.reshape(B)


# speedup vs baseline: 7.3032x; 7.3032x over previous
import jax
import jax.numpy as jnp
from jax import lax
from jax.experimental import pallas as pl
from jax.experimental.pallas import tpu as pltpu
from jax.experimental.pallas import tpu_sc as plsc

B, F, V = 16384, 26, 1000000
F2 = 32
BF2 = B * F2
NC, NS = 2, 16
NW = NC * NS
E = BF2 // NW

XR = BF2 // 128
NBLK = 8
BR = XR // NBLK
SPR = 128 // F2


def _sc_gather_body(idx_hbm, w_hbm, g_hbm, idx_v, g_v, sem):
    wid = lax.axis_index("s") * NC + lax.axis_index("c")
    base = wid * E
    pltpu.sync_copy(idx_hbm.at[pl.ds(base, E)], idx_v)
    pltpu.async_copy(w_hbm.at[idx_v], g_v, sem).wait()
    pltpu.sync_copy(g_v, g_hbm.at[pl.ds(base, E)])


def _tc_reduce_body(b_ref, g_ref, v_ref, m_ref, o_ref):
    p = g_ref[...] * v_ref[...]
    acc = jnp.dot(p, m_ref[...], preferred_element_type=jnp.float32)
    o_ref[...] = jax.nn.sigmoid(acc + b_ref[0])


def kernel(feature_idx, feature_vals, W, b):
    padidx = (jnp.arange(B * (F2 - F), dtype=jnp.int32) % V).reshape(B, F2 - F)
    idx32 = jnp.concatenate(
        [feature_idx.astype(jnp.int32), padidx], axis=1).reshape(BF2)
    vals32 = jnp.pad(feature_vals, ((0, 0), (0, F2 - F))).reshape(BF2)

    mesh = plsc.VectorSubcoreMesh(core_axis_name="c", subcore_axis_name="s")
    gathered = pl.kernel(
        _sc_gather_body,
        out_type=jax.ShapeDtypeStruct((BF2,), jnp.float32),
        mesh=mesh,
        scratch_types=[
            pltpu.VMEM((E,), jnp.int32),
            pltpu.VMEM((E,), jnp.float32),
            pltpu.SemaphoreType.DMA,
        ],
    )(idx32, W)

    c = jnp.arange(128, dtype=jnp.int32)[:, None]
    s = jnp.arange(SPR, dtype=jnp.int32)[None, :]
    m = (c // F2 == s).astype(jnp.float32)

    out2d = pl.pallas_call(
        _tc_reduce_body,
        grid=(NBLK,),
        in_specs=[
            pl.BlockSpec(memory_space=pltpu.SMEM),
            pl.BlockSpec((BR, 128), lambda i: (i, 0)),
            pl.BlockSpec((BR, 128), lambda i: (i, 0)),
            pl.BlockSpec((128, SPR), lambda i: (0, 0)),
        ],
        out_specs=pl.BlockSpec((BR, SPR), lambda i: (i, 0)),
        out_shape=jax.ShapeDtypeStruct((XR, SPR), jnp.float32),
    )(jnp.asarray(b, jnp.float32).reshape(1),
      gathered.reshape(XR, 128), vals32.reshape(XR, 128), m)
    return out2d.reshape(B)


# speedup vs baseline: 8.2832x; 1.1342x over previous
import jax
import jax.numpy as jnp
from jax import lax
from jax.experimental import pallas as pl
from jax.experimental.pallas import tpu as pltpu
from jax.experimental.pallas import tpu_sc as plsc

B, F, V = 16384, 26, 1000000
NC, NS = 2, 16
NW = NC * NS
E = (B * F) // NW

SUP = 13
SEG = 64
NSR = (B * F) // (SUP * 128)
NBLK = 8
BSR = NSR // NBLK


def _sc_gather_body(idx_hbm, w_hbm, g_hbm, idx_v, g_v, sem):
    wid = lax.axis_index("s") * NC + lax.axis_index("c")
    base = wid * E
    pltpu.sync_copy(idx_hbm.at[pl.ds(base, E)], idx_v)
    pltpu.async_copy(w_hbm.at[idx_v], g_v, sem).wait()
    pltpu.sync_copy(g_v, g_hbm.at[pl.ds(base, E)])


def _tc_reduce_body(b_ref, g_ref, v_ref, m_ref, o_ref):
    p = g_ref[...] * v_ref[...]
    acc = jnp.zeros((BSR, SEG), jnp.float32)
    for r in range(SUP):
        acc = acc + jnp.dot(p[:, r, :], m_ref[r],
                            preferred_element_type=jnp.float32)
    o_ref[...] = jax.nn.sigmoid(acc + b_ref[0])


def kernel(feature_idx, feature_vals, W, b):
    idx_flat = feature_idx.astype(jnp.int32).reshape(B * F)

    mesh = plsc.VectorSubcoreMesh(core_axis_name="c", subcore_axis_name="s")
    gathered = pl.kernel(
        _sc_gather_body,
        out_type=jax.ShapeDtypeStruct((B * F,), jnp.float32),
        mesh=mesh,
        scratch_types=[
            pltpu.VMEM((E,), jnp.int32),
            pltpu.VMEM((E,), jnp.float32),
            pltpu.SemaphoreType.DMA,
        ],
    )(idx_flat, W)

    r = jnp.arange(SUP, dtype=jnp.int32)[:, None, None]
    c = jnp.arange(128, dtype=jnp.int32)[None, :, None]
    s = jnp.arange(SEG, dtype=jnp.int32)[None, None, :]
    flat = 128 * r + c
    m = ((flat >= F * s) & (flat < F * s + F)).astype(jnp.float32)

    out2d = pl.pallas_call(
        _tc_reduce_body,
        grid=(NBLK,),
        in_specs=[
            pl.BlockSpec(memory_space=pltpu.SMEM),
            pl.BlockSpec((BSR, SUP, 128), lambda i: (i, 0, 0)),
            pl.BlockSpec((BSR, SUP, 128), lambda i: (i, 0, 0)),
            pl.BlockSpec((SUP, 128, SEG), lambda i: (0, 0, 0)),
        ],
        out_specs=pl.BlockSpec((BSR, SEG), lambda i: (i, 0)),
        out_shape=jax.ShapeDtypeStruct((NSR, SEG), jnp.float32),
    )(jnp.asarray(b, jnp.float32).reshape(1),
      gathered.reshape(NSR, SUP, 128), feature_vals.reshape(NSR, SUP, 128), m)
    return out2d.reshape(B)


# speedup vs baseline: 11.6301x; 1.4041x over previous
import jax
import jax.numpy as jnp
from jax import lax
from jax.experimental import pallas as pl
from jax.experimental.pallas import tpu as pltpu
from jax.experimental.pallas import tpu_sc as plsc

B, F, V = 16384, 26, 1000000
L = 16
NC, NS = 2, 16
NW = NC * NS
ROWS_W = B // NW
E = ROWS_W * F
HALF = E // 2
FH = F // 2


def _sc_body(idx_hbm, vals_hbm, w_hbm, b_hbm, out_hbm,
             idx_v, v_v, g_v, b_v, acc_v, out_v, sem0, sem1):
    wid = lax.axis_index("s") * NC + lax.axis_index("c")
    base = wid * E

    pltpu.sync_copy(idx_hbm.at[pl.ds(base, E)], idx_v)
    cp0 = pltpu.async_copy(
        w_hbm.at[idx_v.at[pl.ds(0, HALF)]], g_v.at[pl.ds(0, HALF)], sem0)
    cp1 = pltpu.async_copy(
        w_hbm.at[idx_v.at[pl.ds(HALF, HALF)]], g_v.at[pl.ds(HALF, HALF)], sem1)
    pltpu.sync_copy(vals_hbm.at[pl.ds(base, E)], v_v)
    pltpu.sync_copy(b_hbm, b_v)

    bvec = b_v[...]

    cp0.wait()

    def part0(sg, carry):
        acc = bvec
        for f in range(FH):
            s = pl.ds(f * ROWS_W + sg * L, L)
            acc = acc + g_v[s] * v_v[s]
        acc_v[pl.ds(sg * L, L)] = acc
        return carry
    lax.fori_loop(0, ROWS_W // L, part0, 0)

    cp1.wait()

    def part1(sg, carry):
        acc = acc_v[pl.ds(sg * L, L)]
        for f in range(FH, F):
            s = pl.ds(f * ROWS_W + sg * L, L)
            acc = acc + g_v[s] * v_v[s]
        out_v[pl.ds(sg * L, L)] = 1.0 / (1.0 + jnp.exp(-acc))
        return carry
    lax.fori_loop(0, ROWS_W // L, part1, 0)

    pltpu.sync_copy(out_v, out_hbm.at[pl.ds(wid * ROWS_W, ROWS_W)])


def kernel(feature_idx, feature_vals, W, b):
    idx_bt = (feature_idx.astype(jnp.int32)
              .reshape(NW, ROWS_W, F).transpose(0, 2, 1).reshape(NW * E))
    vals_bt = feature_vals.reshape(NW, ROWS_W, F).transpose(0, 2, 1).reshape(NW * E)
    b16 = jnp.broadcast_to(jnp.asarray(b, jnp.float32).reshape(()), (L,))

    mesh = plsc.VectorSubcoreMesh(core_axis_name="c", subcore_axis_name="s")
    run = pl.kernel(
        _sc_body,
        out_type=jax.ShapeDtypeStruct((B,), jnp.float32),
        mesh=mesh,
        scratch_types=[
            pltpu.VMEM((E,), jnp.int32),
            pltpu.VMEM((E,), jnp.float32),
            pltpu.VMEM((E,), jnp.float32),
            pltpu.VMEM((L,), jnp.float32),
            pltpu.VMEM((ROWS_W,), jnp.float32),
            pltpu.VMEM((ROWS_W,), jnp.float32),
            pltpu.SemaphoreType.DMA,
            pltpu.SemaphoreType.DMA,
        ],
    )
    return run(idx_bt, vals_bt, W, b16)
